# baseline (speedup 1.0000x reference)
import dataclasses

import jax
import jax.numpy as jnp
from jax import lax
from jax.experimental import pallas as pl
from jax.experimental.pallas import tpu as pltpu
from jax.experimental.pallas import tpu_sc as plsc

N_NODES = 10000
N_EDGES = 320000
D_FEAT = 128
L = 16
NW = 32
EPW = N_EDGES // NW
G = 80
NCHUNK = EPW // G
NBUF = 4


def _sc_body(nf_hbm, src_hbm, dst_hbm, out_hbm,
             src_idx, dst_idx, src_rows, dst_rows, psum, out_v,
             sem_s0, sem_s1, sem_s2, sem_s3, sem_d0, sem_d1, sem_d2, sem_d3):
    sem_s = (sem_s0, sem_s1, sem_s2, sem_s3)
    sem_d = (sem_d0, sem_d1, sem_d2, sem_d3)
    wid = lax.axis_index("s") * 2 + lax.axis_index("c")
    base = wid * EPW

    pltpu.sync_copy(src_hbm.at[pl.ds(base, EPW)], src_idx)
    pltpu.sync_copy(dst_hbm.at[pl.ds(base, EPW)], dst_idx)

    def start_gathers(chunk, b):
        off = pl.multiple_of(chunk * G, 8)
        pltpu.async_copy(nf_hbm.at[src_idx.at[pl.ds(off, G)]], src_rows[b], sem_s[b])
        pltpu.async_copy(nf_hbm.at[dst_idx.at[pl.ds(off, G)]], dst_rows[b], sem_d[b])

    def wait_gathers(b):
        pltpu.make_async_copy(nf_hbm.at[src_idx.at[pl.ds(0, G)]], src_rows[b], sem_s[b]).wait()
        pltpu.make_async_copy(nf_hbm.at[dst_idx.at[pl.ds(0, G)]], dst_rows[b], sem_d[b]).wait()

    def compute(chunk, b):
        sr, dr = src_rows[b], dst_rows[b]

        @pl.loop(0, G, step=4)
        def _edge(e0):
            for de in range(4):
                e = e0 + de
                acc = None
                for j in range(D_FEAT // (2 * L)):
                    sa, sb = plsc.unpack(plsc.bitcast(sr[e, pl.ds(j * L, L)], jnp.bfloat16),
                                         format=plsc.PackFormat.INTERLEAVED)
                    da, db = plsc.unpack(plsc.bitcast(dr[e, pl.ds(j * L, L)], jnp.bfloat16),
                                         format=plsc.PackFormat.INTERLEAVED)
                    term = sa * da + sb * db
                    acc = term if acc is None else acc + term
                psum[e, pl.ds(0, L)] = acc

        out_base = pl.multiple_of(chunk * G, 8)

        @pl.loop(0, G // L)
        def _group(q):
            rows = q * L + lax.iota(jnp.int32, L)
            tot = plsc.load_gather(psum, [rows, jnp.zeros((L,), jnp.int32)])
            for j in range(1, L):
                tot = tot + plsc.load_gather(psum, [rows, jnp.full((L,), j, jnp.int32)])
            out_v[pl.ds(out_base + q * L, L)] = tot

    for b in range(NBUF):
        start_gathers(b, b)

    @pl.loop(0, NCHUNK - 1, step=NBUF)
    def _main(g):
        for b in range(NBUF):
            chunk = g + b
            wait_gathers(b)
            compute(chunk, b)
            nxt = chunk + NBUF

            @pl.when(nxt < NCHUNK)
            def _():
                start_gathers(nxt, b)

    wait_gathers(0)
    compute(NCHUNK - 1, 0)

    pltpu.sync_copy(out_v, out_hbm.at[pl.ds(base, EPW)])


def kernel(node_feature, edge_src, edge_dst):
    mesh = plsc.VectorSubcoreMesh(core_axis_name="c", subcore_axis_name="s")
    cp = pltpu.CompilerParams()
    for fld, val in (("needs_layout_passes", False), ("use_tc_tiling_on_sc", False)):
        if fld in pltpu.CompilerParams.__dataclass_fields__:
            cp = dataclasses.replace(cp, **{fld: val})
    run = pl.kernel(
        _sc_body,
        mesh=mesh,
        compiler_params=cp,
        out_type=jax.ShapeDtypeStruct((N_EDGES,), jnp.float32),
        scratch_types=[
            pltpu.VMEM((EPW,), jnp.int32),
            pltpu.VMEM((EPW,), jnp.int32),
            [pltpu.VMEM((G, D_FEAT // 2), jnp.int32) for _ in range(NBUF)],
            [pltpu.VMEM((G, D_FEAT // 2), jnp.int32) for _ in range(NBUF)],
            pltpu.VMEM((G, L + 1), jnp.float32),
            pltpu.VMEM((EPW,), jnp.float32),
        ] + [pltpu.SemaphoreType.DMA] * (2 * NBUF),
    )
    nf_packed = jax.lax.bitcast_convert_type(
        node_feature.astype(jnp.bfloat16).reshape(N_NODES, D_FEAT // 2, 2),
        jnp.int32)
    return run(nf_packed, edge_src, edge_dst).reshape(N_EDGES, 1)


# speedup vs baseline: 1.5864x; 1.5864x over previous
import dataclasses

import jax
import jax.numpy as jnp
from jax import lax
from jax.experimental import pallas as pl
from jax.experimental.pallas import tpu as pltpu
from jax.experimental.pallas import tpu_sc as plsc

N_NODES = 10000
N_EDGES = 320000
D_FEAT = 128
L = 16
NW = 32
EPW = N_EDGES // NW
G = 80
NCHUNK = EPW // G
NBUF = 4


def _sc_body(nf_hbm, src_hbm, dst_hbm, out_hbm,
             src_idx, dst_idx, src_rows, dst_rows, psum, out_v,
             sem_s0, sem_s1, sem_s2, sem_s3, sem_d0, sem_d1, sem_d2, sem_d3):
    sem_s = (sem_s0, sem_s1, sem_s2, sem_s3)
    sem_d = (sem_d0, sem_d1, sem_d2, sem_d3)
    wid = lax.axis_index("s") * 2 + lax.axis_index("c")
    base = wid * EPW

    pltpu.sync_copy(src_hbm.at[pl.ds(base, EPW)], src_idx)
    pltpu.sync_copy(dst_hbm.at[pl.ds(base, EPW)], dst_idx)

    def start_gathers(chunk, b):
        off = pl.multiple_of(chunk * G, 8)
        pltpu.async_copy(nf_hbm.at[src_idx.at[pl.ds(off, G)]], src_rows[b], sem_s[b])
        pltpu.async_copy(nf_hbm.at[dst_idx.at[pl.ds(off, G)]], dst_rows[b], sem_d[b])

    def wait_gathers(b):
        pltpu.make_async_copy(nf_hbm.at[src_idx.at[pl.ds(0, G)]], src_rows[b], sem_s[b]).wait()
        pltpu.make_async_copy(nf_hbm.at[dst_idx.at[pl.ds(0, G)]], dst_rows[b], sem_d[b]).wait()

    def compute(chunk, b):
        sr, dr = src_rows[b], dst_rows[b]

        @plsc.parallel_loop(0, G, step=1, unroll=4)
        def _edge(e):
            acc = None
            for j in range(D_FEAT // (2 * L)):
                sa, sb = plsc.unpack(plsc.bitcast(sr[e, pl.ds(j * L, L)], jnp.bfloat16),
                                     format=plsc.PackFormat.INTERLEAVED)
                da, db = plsc.unpack(plsc.bitcast(dr[e, pl.ds(j * L, L)], jnp.bfloat16),
                                     format=plsc.PackFormat.INTERLEAVED)
                term = sa * da + sb * db
                acc = term if acc is None else acc + term
            psum[e, pl.ds(0, L)] = acc

        out_base = pl.multiple_of(chunk * G, 8)

        @plsc.parallel_loop(0, G // L, step=1, unroll=5)
        def _group(q):
            rows = q * L + lax.iota(jnp.int32, L)
            tot = plsc.load_gather(psum, [rows, jnp.zeros((L,), jnp.int32)])
            for j in range(1, L):
                tot = tot + plsc.load_gather(psum, [rows, jnp.full((L,), j, jnp.int32)])
            out_v[pl.ds(out_base + q * L, L)] = tot

    for b in range(NBUF):
        start_gathers(b, b)

    @pl.loop(0, NCHUNK - 1, step=NBUF)
    def _main(g):
        for b in range(NBUF):
            chunk = g + b
            wait_gathers(b)
            compute(chunk, b)
            nxt = chunk + NBUF

            @pl.when(nxt < NCHUNK)
            def _():
                start_gathers(nxt, b)

    wait_gathers(0)
    compute(NCHUNK - 1, 0)

    pltpu.sync_copy(out_v, out_hbm.at[pl.ds(base, EPW)])


def kernel(node_feature, edge_src, edge_dst):
    mesh = plsc.VectorSubcoreMesh(core_axis_name="c", subcore_axis_name="s")
    cp = pltpu.CompilerParams()
    for fld, val in (("needs_layout_passes", False), ("use_tc_tiling_on_sc", False)):
        if fld in pltpu.CompilerParams.__dataclass_fields__:
            cp = dataclasses.replace(cp, **{fld: val})
    run = pl.kernel(
        _sc_body,
        mesh=mesh,
        compiler_params=cp,
        out_type=jax.ShapeDtypeStruct((N_EDGES,), jnp.float32),
        scratch_types=[
            pltpu.VMEM((EPW,), jnp.int32),
            pltpu.VMEM((EPW,), jnp.int32),
            [pltpu.VMEM((G, D_FEAT // 2), jnp.int32) for _ in range(NBUF)],
            [pltpu.VMEM((G, D_FEAT // 2), jnp.int32) for _ in range(NBUF)],
            pltpu.VMEM((G, L + 1), jnp.float32),
            pltpu.VMEM((EPW,), jnp.float32),
        ] + [pltpu.SemaphoreType.DMA] * (2 * NBUF),
    )
    nf_packed = jax.lax.bitcast_convert_type(
        node_feature.astype(jnp.bfloat16).reshape(N_NODES, D_FEAT // 2, 2),
        jnp.int32)
    return run(nf_packed, edge_src, edge_dst).reshape(N_EDGES, 1)
